# baseline (speedup 1.0000x reference)
import functools
import jax
import jax.numpy as jnp
from jax import lax
from jax.experimental import pallas as pl
from jax.experimental.pallas import tpu as pltpu
from jax.experimental.pallas import tpu_sc as plsc

_B = 8192
_D = 2048
_K = 64
_BM = 1024
_NW = 32
_BPW = _B // _NW


def _selector(aux_ref, x_ref, u_ref, tb_ref, b1_ref, w2_ref, b2_ref,
              db1_ref, db2_ref, idx_ref, dec_out_ref, dec_ref):
    @pl.when(pl.program_id(0) == 0)
    def _():
        t = tb_ref[...]
        h2 = jnp.dot(t, aux_ref[2048:3072, 0:32],
                     preferred_element_type=jnp.float32)
        h2 = jnp.maximum(h2 + db1_ref[...], 0.0)
        dec_ref[...] = jnp.dot(h2, aux_ref[3072:3104, 0:1],
                               preferred_element_type=jnp.float32) + db2_ref[...]

    x = x_ref[...]
    h = jnp.dot(x, aux_ref[0:2048, 0:64],
                preferred_element_type=jnp.float32)
    h = jnp.maximum(h + b1_ref[...], 0.0)
    logits = jnp.dot(h, w2_ref[...],
                     preferred_element_type=jnp.float32) + b2_ref[...]
    boosted = logits + 1.0
    g = -jnp.log(-jnp.log(u_ref[...]))
    s = boosted + g

    m = jnp.max(s, axis=-1, keepdims=True)
    iota = lax.broadcasted_iota(jnp.int32, s.shape, 1)
    first = jnp.min(jnp.where(s == m, iota, _K), axis=-1, keepdims=True)
    idx_ref[...] = first.reshape(_BM)
    dec_out_ref[...] = dec_ref[...].reshape(_K)


def _gather16(vec, ids):
    dn = lax.GatherDimensionNumbers(offset_dims=(), collapsed_slice_dims=(0,),
                                    start_index_map=(0,))
    return lax.gather(vec, ids[:, None], dn, slice_sizes=(1,),
                      mode=lax.GatherScatterMode.PROMISE_IN_BOUNDS)


def _sc_gather(idx_hbm, table_hbm, out_hbm, idx_v, table_v, out_v):
    wid = lax.axis_index("s") * 2 + lax.axis_index("c")
    base = wid * _BPW
    pltpu.sync_copy(table_hbm, table_v)
    pltpu.sync_copy(idx_hbm.at[pl.ds(base, _BPW)], idx_v)
    t = [table_v[pl.ds(16 * q, 16)] for q in range(4)]
    for i in range(_BPW // 16):
        ids = idx_v[pl.ds(i * 16, 16)]
        val = _gather16(t[0], ids & 15)
        for q in range(1, 4):
            gq = _gather16(t[q], ids & 15)
            val = jnp.where(ids >= 16 * q, gq, val)
        out_v[pl.ds(i * 16, 16)] = val


def kernel(x, gumbel_u, thought_bank, sel_w1, sel_b1, sel_w2, sel_b2,
           dec_w1, dec_b1, dec_w2, dec_b2):
    aux = jnp.concatenate([
        jnp.pad(sel_w1, ((0, 0), (0, 64))),
        jnp.pad(dec_w1, ((0, 0), (0, 96))),
        jnp.pad(dec_w2, ((0, 0), (0, 127))),
    ], axis=0)
    grid = (_B // _BM,)
    idx, decoded = pl.pallas_call(
        _selector,
        grid=grid,
        in_specs=[
            pl.BlockSpec((3104, 128), lambda i: (0, 0)),
            pl.BlockSpec((_BM, _D), lambda i: (i, 0)),
            pl.BlockSpec((_BM, _K), lambda i: (i, 0)),
            pl.BlockSpec((_K, 1024), lambda i: (0, 0)),
            pl.BlockSpec((1, _K), lambda i: (0, 0)),
            pl.BlockSpec((_K, _K), lambda i: (0, 0)),
            pl.BlockSpec((1, _K), lambda i: (0, 0)),
            pl.BlockSpec((1, 32), lambda i: (0, 0)),
            pl.BlockSpec((1, 1), lambda i: (0, 0)),
        ],
        out_specs=[
            pl.BlockSpec((_BM,), lambda i: (i,)),
            pl.BlockSpec((_K,), lambda i: (0,)),
        ],
        out_shape=[
            jax.ShapeDtypeStruct((_B,), jnp.int32),
            jax.ShapeDtypeStruct((_K,), jnp.float32),
        ],
        scratch_shapes=[pltpu.VMEM((_K, 1), jnp.float32)],
    )(aux, x, gumbel_u, thought_bank, sel_b1.reshape(1, _K), sel_w2,
      sel_b2.reshape(1, _K), dec_b1.reshape(1, 32), dec_b2.reshape(1, 1))

    mesh = plsc.VectorSubcoreMesh(core_axis_name="c", subcore_axis_name="s")
    gather = functools.partial(
        pl.kernel,
        mesh=mesh,
        out_type=jax.ShapeDtypeStruct((_B,), jnp.float32),
        scratch_types=[
            pltpu.VMEM((_BPW,), jnp.int32),
            pltpu.VMEM((_K,), jnp.float32),
            pltpu.VMEM((_BPW,), jnp.float32),
        ],
    )(_sc_gather)
    return gather(idx, decoded)


# speedup vs baseline: 1.5329x; 1.5329x over previous
import jax
import jax.numpy as jnp
from jax import lax
from jax.experimental import pallas as pl
from jax.experimental.pallas import tpu as pltpu

_B = 8192
_D = 2048
_K = 64
_BM = 1024


def _fused(aux_ref, x_ref, u_ref, tb_ref, b1_ref, w2_ref, b2_ref,
           db1_ref, db2_ref, out_ref, dec_ref):
    @pl.when(pl.program_id(0) == 0)
    def _():
        t = tb_ref[...]
        h2 = jnp.dot(t, aux_ref[2048:3072, 0:32],
                     preferred_element_type=jnp.float32)
        h2 = jnp.maximum(h2 + db1_ref[...], 0.0)
        dec_ref[...] = jnp.dot(h2, aux_ref[3072:3104, 0:1],
                               preferred_element_type=jnp.float32) + db2_ref[...]

    x = x_ref[...]
    h = jnp.dot(x.astype(jnp.bfloat16),
                aux_ref[0:2048, 0:64].astype(jnp.bfloat16),
                preferred_element_type=jnp.float32)
    h = jnp.maximum(h + b1_ref[...], 0.0)
    logits = jnp.dot(h, w2_ref[...],
                     preferred_element_type=jnp.float32) + b2_ref[...]
    boosted = logits + 1.0
    g = -jnp.log(-jnp.log(u_ref[...]))
    s = boosted + g

    m = jnp.max(s, axis=-1, keepdims=True)
    iota = lax.broadcasted_iota(jnp.int32, s.shape, 1)
    first = jnp.min(jnp.where(s == m, iota, _K), axis=-1, keepdims=True)
    onehot = (iota == first).astype(jnp.float32)

    out = jnp.dot(onehot, dec_ref[...],
                  preferred_element_type=jnp.float32)
    out_ref[...] = out.reshape(_BM)


def kernel(x, gumbel_u, thought_bank, sel_w1, sel_b1, sel_w2, sel_b2,
           dec_w1, dec_b1, dec_w2, dec_b2):
    aux = jnp.concatenate([
        jnp.pad(sel_w1, ((0, 0), (0, 64))),
        jnp.pad(dec_w1, ((0, 0), (0, 96))),
        jnp.pad(dec_w2, ((0, 0), (0, 127))),
    ], axis=0)
    grid = (_B // _BM,)
    out = pl.pallas_call(
        _fused,
        grid=grid,
        in_specs=[
            pl.BlockSpec((3104, 128), lambda i: (0, 0)),
            pl.BlockSpec((_BM, _D), lambda i: (i, 0)),
            pl.BlockSpec((_BM, _K), lambda i: (i, 0)),
            pl.BlockSpec((_K, 1024), lambda i: (0, 0)),
            pl.BlockSpec((1, _K), lambda i: (0, 0)),
            pl.BlockSpec((_K, _K), lambda i: (0, 0)),
            pl.BlockSpec((1, _K), lambda i: (0, 0)),
            pl.BlockSpec((1, 32), lambda i: (0, 0)),
            pl.BlockSpec((1, 1), lambda i: (0, 0)),
        ],
        out_specs=pl.BlockSpec((_BM,), lambda i: (i,)),
        out_shape=jax.ShapeDtypeStruct((_B,), jnp.float32),
        scratch_shapes=[pltpu.VMEM((_K, 1), jnp.float32)],
    )(aux, x, gumbel_u, thought_bank, sel_b1.reshape(1, _K), sel_w2,
      sel_b2.reshape(1, _K), dec_b1.reshape(1, 32), dec_b2.reshape(1, 1))
    return out
